# Initial kernel scaffold; baseline (speedup 1.0000x reference)
import jax
import jax.numpy as jnp
from jax.experimental import pallas as pl


def kernel(lhs, rel):
    raise NotImplementedError("write your pallas kernel here")


# speedup vs baseline: 1.7963x; 1.7963x over previous
import jax
import jax.numpy as jnp
from jax.experimental import pallas as pl


def _complex_body(lhs_ref, rel_ref, out_ref):
    lhs = lhs_ref[...]
    rel = rel_ref[...]
    r = lhs.shape[-1] // 2
    l0, l1 = lhs[:, :r], lhs[:, r:]
    r0, r1 = rel[:, :r], rel[:, r:]
    out_ref[:, :r] = l0 * r0 - l1 * r1
    out_ref[:, r:] = l0 * r1 + l1 * r0


def kernel(lhs, rel):
    B, D = lhs.shape
    blk = 2048
    return pl.pallas_call(
        _complex_body,
        grid=(B // blk,),
        in_specs=[
            pl.BlockSpec((blk, D), lambda i: (i, 0)),
            pl.BlockSpec((blk, D), lambda i: (i, 0)),
        ],
        out_specs=pl.BlockSpec((blk, D), lambda i: (i, 0)),
        out_shape=jax.ShapeDtypeStruct((B, D), lhs.dtype),
    )(lhs, rel)
